# baseline (speedup 1.0000x reference)
import math

import jax
import jax.numpy as jnp
import numpy as np
from jax import lax
from jax.experimental import pallas as pl
from jax.experimental.pallas import tpu as pltpu
from jax.experimental.pallas import tpu_sc as plsc

L_SP = 24
L_T = 24
FDIM = 2
LOG2_T = 20
TBL = 1 << LOG2_T
MASK = TBL - 1
SP_RES = np.floor(16.0 * ((4096.0 / 16.0) ** (np.arange(L_SP) / (L_SP - 1)))).astype(np.float32)
T_RES = np.floor(8.0 * ((8192.0 / 8.0) ** (np.arange(L_T) / (L_T - 1)))).astype(np.float32)
P2 = 2654435761
P3 = 805459861

NC = 2
NS = 16
NW = NC * NS
PG = 128
LVL_Q = 6
NSPQ = LVL_Q * 8 * FDIM * PG
L_T_LUT = 21
L_T_HBM = L_T - L_T_LUT
NTH = L_T_HBM * 2 * FDIM * PG
T_E = [int(v) + 2 for v in T_RES[:L_T_LUT]]
T_REG = [-(-e // 16) * 16 for e in T_E]
T_WBASE = [0]
for _r in T_REG:
    T_WBASE.append(T_WBASE[-1] + 2 * _r)
LUT_WORDS = T_WBASE[-1]
D_ENC = (L_SP + L_T) * FDIM
TWORDS = TBL * FDIM
NSPLIT = 4


def _prep_body(ct_ref, o_ref):
    deg = math.pi / 180.0
    lat = ct_ref[0, :] * deg
    lon = ct_ref[1, :] * deg
    elev = ct_ref[2, :]
    t = ct_ref[3, :]
    R = 6371000.0
    r = R + elev
    s = 2.0 * (R + 10000.0)
    cl = jnp.cos(lat)
    x01 = r * cl * jnp.cos(lon) / s + 0.5
    y01 = r * cl * jnp.sin(lon) / s + 0.5
    z01 = r * jnp.sin(lat) / s + 0.5
    o_ref[...] = jnp.stack([x01, y01, z01, t], axis=0)


def _mlp_body(f_ref, w1_ref, b1_ref, w2_ref, b2_ref, w3_ref, b3_ref, o_ref):
    f = f_ref[...]
    dn = (((0,), (0,)), ((), ()))
    h1 = lax.dot_general(w1_ref[...], f, dn, preferred_element_type=jnp.float32)
    h1 = jnp.maximum(h1 + b1_ref[...][:, None], 0.0)
    h2 = lax.dot_general(w2_ref[...], h1, dn, preferred_element_type=jnp.float32)
    h2 = jnp.maximum(h2 + b2_ref[...][:, None], 0.0)
    o = lax.dot_general(w3_ref[...], h2, dn, preferred_element_type=jnp.float32)
    o = o + b3_ref[...][:, None]
    o_ref[...] = 1.0 / (1.0 + jnp.exp(-o))


def _phys_word(h):
    return (((h >> 7) * 256) | (h & 127)).astype(jnp.int32)


def _sc_body(xyzt_hbm, tsp_hbm, tt_hbm, feats_hbm,
             isp_v0, isp_v1, vsp_v0, vsp_v1, wsp_v, it_v, vt_v, wt_v,
             f_v, xyzt_v, lut_v, res_sp_s, res_t_s, lutw_s, sem0, sem1, sem_t):
    wid = lax.axis_index("s") * NC + lax.axis_index("c")
    chunks_per_tile = xyzt_hbm.shape[1] // (NW * PG)
    p2u = jnp.uint32(P2)
    p3u = jnp.uint32(P3)
    msku = jnp.uint32(MASK)

    for l in range(L_SP):
        res_sp_s[l] = jnp.float32(float(SP_RES[l]))
    for l in range(L_T):
        res_t_s[l] = jnp.float32(float(T_RES[l]))
    for l in range(L_T_LUT):
        lutw_s[l] = jnp.int32(T_WBASE[l])
    iota16 = lax.iota(jnp.int32, 16)

    for l in range(L_T_LUT):
        base = l * TWORDS

        def bgrp(k, carry, base=base):
            j = k * 16 + iota16
            u = j.astype(jnp.uint32) * p2u
            w = _phys_word(u & msku) + base
            jj = k * 32 + iota16 * 2
            plsc.store_scatter(isp_v0, [jj], w)
            plsc.store_scatter(isp_v0, [jj + 1], w + 128)
            return carry

        lax.fori_loop(0, T_REG[l] // 16, bgrp, 0, unroll=False)
        pltpu.async_copy(
            tt_hbm.at[isp_v0.at[pl.ds(0, 2 * T_REG[l])]],
            lut_v.at[pl.ds(T_WBASE[l], 2 * T_REG[l])], sem_t).wait()

    def hash_spatial_q(q, isp_v):
        def grp(g, carry):
            lane0 = g * 16
            x = xyzt_v[0, pl.ds(lane0, 16)]
            y = xyzt_v[1, pl.ds(lane0, 16)]
            z = xyzt_v[2, pl.ds(lane0, 16)]

            def lvl(ll, carry2):
                l = q * LVL_Q + ll
                res = res_sp_s[l]
                px = x * res
                py = y * res
                pz = z * res
                ix = px.astype(jnp.int32)
                iy = py.astype(jnp.int32)
                iz = pz.astype(jnp.int32)
                fx = px - ix.astype(jnp.float32)
                fy = py - iy.astype(jnp.float32)
                fz = pz - iz.astype(jnp.float32)
                ixu = ix.astype(jnp.uint32)
                iyu = iy.astype(jnp.uint32)
                izu = iz.astype(jnp.uint32)
                hx = (ixu, ixu + jnp.uint32(1))
                hy0 = iyu * p2u
                hy = (hy0, hy0 + p2u)
                hz0 = izu * p3u
                hz = (hz0, hz0 + p3u)
                wx = (1.0 - fx, fx)
                wy = (1.0 - fy, fy)
                wz = (1.0 - fz, fz)
                base = l * TWORDS
                for c in range(8):
                    oi, oj, ok = c >> 2, (c >> 1) & 1, c & 1
                    hh = (hx[oi] ^ hy[oj] ^ hz[ok]) & msku
                    w = _phys_word(hh) + base
                    pos = ((ll * 8 + c) * 2) * PG + lane0
                    isp_v[pl.ds(pos, 16)] = w
                    isp_v[pl.ds(pos + PG, 16)] = w + 128
                    wsp_v[q * 48 + ll * 8 + c, pl.ds(lane0, 16)] = \
                        wx[oi] * wy[oj] * wz[ok]
                return carry2

            lax.fori_loop(0, LVL_Q, lvl, 0, unroll=False)
            return carry

        lax.fori_loop(0, PG // 16, grp, 0, unroll=False)

    def hash_temporal():
        def grp(g, carry):
            lane0 = g * 16
            t = xyzt_v[3, pl.ds(lane0, 16)]

            def lvl(ll, carry2):
                l = L_T_LUT + ll
                res = res_t_s[l]
                pt = t * res
                i0 = pt.astype(jnp.int32)
                ft = pt - i0.astype(jnp.float32)
                u = i0.astype(jnp.uint32) * p2u
                h0 = u & msku
                h1 = (u + p2u) & msku
                base = l * TWORDS
                w0 = _phys_word(h0) + base
                w1 = _phys_word(h1) + base
                pos = (4 * ll) * PG + lane0
                it_v[pl.ds(pos, 16)] = w0
                it_v[pl.ds(pos + PG, 16)] = w0 + 128
                it_v[pl.ds(pos + 2 * PG, 16)] = w1
                it_v[pl.ds(pos + 3 * PG, 16)] = w1 + 128
                wt_v[2 * ll, pl.ds(lane0, 16)] = 1.0 - ft
                wt_v[2 * ll + 1, pl.ds(lane0, 16)] = ft
                return carry2

            lax.fori_loop(0, L_T_HBM, lvl, 0, unroll=False)
            return carry

        lax.fori_loop(0, PG // 16, grp, 0, unroll=False)

    def reduce_spatial_q(q, vsp_v):
        def grp(g, carry):
            lane0 = g * 16

            def lvl(ll, carry2):
                l = q * LVL_Q + ll
                acc0 = jnp.zeros((16,), jnp.float32)
                acc1 = jnp.zeros((16,), jnp.float32)
                for c in range(8):
                    wv = wsp_v[q * 48 + ll * 8 + c, pl.ds(lane0, 16)]
                    pos = ((ll * 8 + c) * 2) * PG + lane0
                    acc0 = acc0 + vsp_v[pl.ds(pos, 16)] * wv
                    acc1 = acc1 + vsp_v[pl.ds(pos + PG, 16)] * wv
                f_v[2 * l, pl.ds(lane0, 16)] = acc0
                f_v[2 * l + 1, pl.ds(lane0, 16)] = acc1
                return carry2

            lax.fori_loop(0, LVL_Q, lvl, 0, unroll=False)
            return carry

        lax.fori_loop(0, PG // 16, grp, 0, unroll=False)

    def reduce_temporal():
        def grp(g, carry):
            lane0 = g * 16

            def lvl(ll, carry2):
                l = L_T_LUT + ll
                w0 = wt_v[2 * ll, pl.ds(lane0, 16)]
                w1 = wt_v[2 * ll + 1, pl.ds(lane0, 16)]
                pos = (4 * ll) * PG + lane0
                a0 = vt_v[pl.ds(pos, 16)] * w0 + vt_v[pl.ds(pos + 2 * PG, 16)] * w1
                a1 = vt_v[pl.ds(pos + PG, 16)] * w0 + vt_v[pl.ds(pos + 3 * PG, 16)] * w1
                f_v[2 * L_SP + 2 * l, pl.ds(lane0, 16)] = a0
                f_v[2 * L_SP + 2 * l + 1, pl.ds(lane0, 16)] = a1
                return carry2

            lax.fori_loop(0, L_T_HBM, lvl, 0, unroll=False)
            return carry

        lax.fori_loop(0, PG // 16, grp, 0, unroll=False)

    def lut_temporal():
        def grp(g, carry):
            lane0 = g * 16
            t = xyzt_v[3, pl.ds(lane0, 16)]

            def lvl(l, carry2):
                res = res_t_s[l]
                wb = lutw_s[l]
                pt = t * res
                i0 = pt.astype(jnp.int32)
                ft = pt - i0.astype(jnp.float32)
                pp = wb + 2 * i0
                v00 = plsc.load_gather(lut_v, [pp])
                v01 = plsc.load_gather(lut_v, [pp + 1])
                v10 = plsc.load_gather(lut_v, [pp + 2])
                v11 = plsc.load_gather(lut_v, [pp + 3])
                w0 = 1.0 - ft
                f_v[2 * L_SP + 2 * l, pl.ds(lane0, 16)] = v00 * w0 + v10 * ft
                f_v[2 * L_SP + 2 * l + 1, pl.ds(lane0, 16)] = v01 * w0 + v11 * ft
                return carry2

            lax.fori_loop(0, L_T_LUT, lvl, 0, unroll=False)
            return carry

        lax.fori_loop(0, PG // 16, grp, 0, unroll=False)

    def chunk(ci, carry):
        gidx = wid * chunks_per_tile + ci
        pbase = gidx * PG
        pltpu.sync_copy(xyzt_hbm.at[:, pl.ds(pbase, PG)], xyzt_v)

        hash_temporal()
        dt = pltpu.async_copy(tt_hbm.at[it_v], vt_v, sem_t)
        hash_spatial_q(0, isp_v0)
        d0 = pltpu.async_copy(tsp_hbm.at[isp_v0], vsp_v0, sem0)
        hash_spatial_q(1, isp_v1)
        d1 = pltpu.async_copy(tsp_hbm.at[isp_v1], vsp_v1, sem1)

        d0.wait()
        reduce_spatial_q(0, vsp_v0)
        hash_spatial_q(2, isp_v0)
        d0b = pltpu.async_copy(tsp_hbm.at[isp_v0], vsp_v0, sem0)

        d1.wait()
        reduce_spatial_q(1, vsp_v1)
        hash_spatial_q(3, isp_v1)
        d1b = pltpu.async_copy(tsp_hbm.at[isp_v1], vsp_v1, sem1)

        lut_temporal()
        dt.wait()
        reduce_temporal()

        d0b.wait()
        reduce_spatial_q(2, vsp_v0)
        d1b.wait()
        reduce_spatial_q(3, vsp_v1)

        pltpu.sync_copy(f_v, feats_hbm.at[:, pl.ds(pbase, PG)])
        return carry

    lax.fori_loop(0, chunks_per_tile, chunk, 0, unroll=False)


def kernel(coords, spatial_table, temporal_table, W1, b1, W2, b2, W3, b3):
    B = coords.shape[0]
    BH = B // NSPLIT
    assert B % (NSPLIT * PG * NW) == 0

    coords_t = coords.T

    tsp = (spatial_table.reshape(L_SP, TBL // 128, 128, FDIM)
           .transpose(0, 1, 3, 2).reshape(L_SP * TBL * FDIM))
    tt = (temporal_table.reshape(L_T, TBL // 128, 128, FDIM)
          .transpose(0, 1, 3, 2).reshape(L_T * TBL * FDIM))

    prep = pl.pallas_call(
        _prep_body,
        grid=(BH // 512,),
        in_specs=[pl.BlockSpec((4, 512), lambda i: (0, i))],
        out_specs=pl.BlockSpec((4, 512), lambda i: (0, i)),
        out_shape=jax.ShapeDtypeStruct((4, BH), jnp.float32),
    )

    mesh = plsc.VectorSubcoreMesh(core_axis_name="c", subcore_axis_name="s")
    enc = pl.kernel(
        _sc_body,
        out_type=jax.ShapeDtypeStruct((D_ENC, BH), jnp.float32),
        mesh=mesh,
        scratch_types=[
            pltpu.VMEM((NSPQ,), jnp.int32),
            pltpu.VMEM((NSPQ,), jnp.int32),
            pltpu.VMEM((NSPQ,), jnp.float32),
            pltpu.VMEM((NSPQ,), jnp.float32),
            pltpu.VMEM((L_SP * 8, PG), jnp.float32),
            pltpu.VMEM((NTH,), jnp.int32),
            pltpu.VMEM((NTH,), jnp.float32),
            pltpu.VMEM((L_T_HBM * 2, PG), jnp.float32),
            pltpu.VMEM((D_ENC, PG), jnp.float32),
            pltpu.VMEM((4, PG), jnp.float32),
            pltpu.VMEM((LUT_WORDS,), jnp.float32),
            pltpu.SMEM((L_SP,), jnp.float32),
            pltpu.SMEM((L_T,), jnp.float32),
            pltpu.SMEM((L_T_LUT,), jnp.int32),
            pltpu.SemaphoreType.DMA,
            pltpu.SemaphoreType.DMA,
            pltpu.SemaphoreType.DMA,
        ],
        compiler_params=pltpu.CompilerParams(needs_layout_passes=False),
    )

    mlp = pl.pallas_call(
        _mlp_body,
        grid=(BH // 512,),
        in_specs=[
            pl.BlockSpec((D_ENC, 512), lambda i: (0, i)),
            pl.BlockSpec((D_ENC, 256), lambda i: (0, 0)),
            pl.BlockSpec((256,), lambda i: (0,)),
            pl.BlockSpec((256, 256), lambda i: (0, 0)),
            pl.BlockSpec((256,), lambda i: (0,)),
            pl.BlockSpec((256, 3), lambda i: (0, 0)),
            pl.BlockSpec((3,), lambda i: (0,)),
        ],
        out_specs=pl.BlockSpec((3, 512), lambda i: (0, i)),
        out_shape=jax.ShapeDtypeStruct((3, BH), jnp.float32),
    )

    outs = []
    for hb in range(NSPLIT):
        xyzt = prep(lax.slice_in_dim(coords_t, hb * BH, (hb + 1) * BH, axis=1))
        feats = enc(xyzt, tsp, tt)
        outs.append(mlp(feats, W1, b1, W2, b2, W3, b3))
    return jnp.concatenate([o.T for o in outs], axis=0)


# speedup vs baseline: 1.0365x; 1.0365x over previous
import math

import jax
import jax.numpy as jnp
import numpy as np
from jax import lax
from jax.experimental import pallas as pl
from jax.experimental.pallas import tpu as pltpu
from jax.experimental.pallas import tpu_sc as plsc

L_SP = 24
L_T = 24
FDIM = 2
LOG2_T = 20
TBL = 1 << LOG2_T
MASK = TBL - 1
SP_RES = np.floor(16.0 * ((4096.0 / 16.0) ** (np.arange(L_SP) / (L_SP - 1)))).astype(np.float32)
T_RES = np.floor(8.0 * ((8192.0 / 8.0) ** (np.arange(L_T) / (L_T - 1)))).astype(np.float32)
P2 = 2654435761
P3 = 805459861

NC = 2
NS = 16
NW = NC * NS
PG = 128
LVL_Q = 6
NSPQ = LVL_Q * 8 * FDIM * PG
L_T_LUT = 21
L_T_HBM = L_T - L_T_LUT
NTH = L_T_HBM * 2 * FDIM * PG
T_E = [int(v) + 2 for v in T_RES[:L_T_LUT]]
T_REG = [-(-e // 16) * 16 for e in T_E]
T_WBASE = [0]
for _r in T_REG:
    T_WBASE.append(T_WBASE[-1] + 2 * _r)
LUT_WORDS = T_WBASE[-1]
D_ENC = (L_SP + L_T) * FDIM
TWORDS = TBL * FDIM
NSPLIT = 2


def _prep_body(ct_ref, o_ref):
    deg = math.pi / 180.0
    lat = ct_ref[0, :] * deg
    lon = ct_ref[1, :] * deg
    elev = ct_ref[2, :]
    t = ct_ref[3, :]
    R = 6371000.0
    r = R + elev
    s = 2.0 * (R + 10000.0)
    cl = jnp.cos(lat)
    x01 = r * cl * jnp.cos(lon) / s + 0.5
    y01 = r * cl * jnp.sin(lon) / s + 0.5
    z01 = r * jnp.sin(lat) / s + 0.5
    o_ref[...] = jnp.stack([x01, y01, z01, t], axis=0)


def _mlp_body(f_ref, w1_ref, b1_ref, w2_ref, b2_ref, w3_ref, b3_ref, o_ref):
    f = f_ref[...]
    dn = (((0,), (0,)), ((), ()))
    h1 = lax.dot_general(w1_ref[...], f, dn, preferred_element_type=jnp.float32)
    h1 = jnp.maximum(h1 + b1_ref[...][:, None], 0.0)
    h2 = lax.dot_general(w2_ref[...], h1, dn, preferred_element_type=jnp.float32)
    h2 = jnp.maximum(h2 + b2_ref[...][:, None], 0.0)
    o = lax.dot_general(w3_ref[...], h2, dn, preferred_element_type=jnp.float32)
    o = o + b3_ref[...][:, None]
    o_ref[...] = 1.0 / (1.0 + jnp.exp(-o))


def _phys_word(h):
    return (((h >> 7) * 256) | (h & 127)).astype(jnp.int32)


def _sc_body(xyzt_hbm, tsp_hbm, tt_hbm, feats_hbm,
             isp_v0, isp_v1, vsp_v0, vsp_v1, wsp_v, it_v, vt_v, wt_v,
             f_v, xyzt_v, lut_v, res_sp_s, res_t_s, lutw_s, sem0, sem1, sem_t):
    wid = lax.axis_index("s") * NC + lax.axis_index("c")
    chunks_per_tile = xyzt_hbm.shape[1] // (NW * PG)
    p2u = jnp.uint32(P2)
    p3u = jnp.uint32(P3)
    msku = jnp.uint32(MASK)

    for l in range(L_SP):
        res_sp_s[l] = jnp.float32(float(SP_RES[l]))
    for l in range(L_T):
        res_t_s[l] = jnp.float32(float(T_RES[l]))
    for l in range(L_T_LUT):
        lutw_s[l] = jnp.int32(T_WBASE[l])
    iota16 = lax.iota(jnp.int32, 16)

    for l in range(L_T_LUT):
        base = l * TWORDS

        def bgrp(k, carry, base=base):
            j = k * 16 + iota16
            u = j.astype(jnp.uint32) * p2u
            w = _phys_word(u & msku) + base
            jj = k * 32 + iota16 * 2
            plsc.store_scatter(isp_v0, [jj], w)
            plsc.store_scatter(isp_v0, [jj + 1], w + 128)
            return carry

        lax.fori_loop(0, T_REG[l] // 16, bgrp, 0, unroll=False)
        pltpu.async_copy(
            tt_hbm.at[isp_v0.at[pl.ds(0, 2 * T_REG[l])]],
            lut_v.at[pl.ds(T_WBASE[l], 2 * T_REG[l])], sem_t).wait()

    def hash_spatial_q(q, isp_v):
        def grp(g, carry):
            lane0 = g * 16
            x = xyzt_v[0, pl.ds(lane0, 16)]
            y = xyzt_v[1, pl.ds(lane0, 16)]
            z = xyzt_v[2, pl.ds(lane0, 16)]

            def lvl(ll, carry2):
                l = q * LVL_Q + ll
                res = res_sp_s[l]
                px = x * res
                py = y * res
                pz = z * res
                ix = px.astype(jnp.int32)
                iy = py.astype(jnp.int32)
                iz = pz.astype(jnp.int32)
                fx = px - ix.astype(jnp.float32)
                fy = py - iy.astype(jnp.float32)
                fz = pz - iz.astype(jnp.float32)
                ixu = ix.astype(jnp.uint32)
                iyu = iy.astype(jnp.uint32)
                izu = iz.astype(jnp.uint32)
                hx = (ixu, ixu + jnp.uint32(1))
                hy0 = iyu * p2u
                hy = (hy0, hy0 + p2u)
                hz0 = izu * p3u
                hz = (hz0, hz0 + p3u)
                wx = (1.0 - fx, fx)
                wy = (1.0 - fy, fy)
                wz = (1.0 - fz, fz)
                base = l * TWORDS
                for c in range(8):
                    oi, oj, ok = c >> 2, (c >> 1) & 1, c & 1
                    hh = (hx[oi] ^ hy[oj] ^ hz[ok]) & msku
                    w = _phys_word(hh) + base
                    pos = ((ll * 8 + c) * 2) * PG + lane0
                    isp_v[pl.ds(pos, 16)] = w
                    isp_v[pl.ds(pos + PG, 16)] = w + 128
                    wsp_v[q * 48 + ll * 8 + c, pl.ds(lane0, 16)] = \
                        wx[oi] * wy[oj] * wz[ok]
                return carry2

            lax.fori_loop(0, LVL_Q, lvl, 0, unroll=False)
            return carry

        lax.fori_loop(0, PG // 16, grp, 0, unroll=False)

    def hash_temporal():
        def grp(g, carry):
            lane0 = g * 16
            t = xyzt_v[3, pl.ds(lane0, 16)]

            def lvl(ll, carry2):
                l = L_T_LUT + ll
                res = res_t_s[l]
                pt = t * res
                i0 = pt.astype(jnp.int32)
                ft = pt - i0.astype(jnp.float32)
                u = i0.astype(jnp.uint32) * p2u
                h0 = u & msku
                h1 = (u + p2u) & msku
                base = l * TWORDS
                w0 = _phys_word(h0) + base
                w1 = _phys_word(h1) + base
                pos = (4 * ll) * PG + lane0
                it_v[pl.ds(pos, 16)] = w0
                it_v[pl.ds(pos + PG, 16)] = w0 + 128
                it_v[pl.ds(pos + 2 * PG, 16)] = w1
                it_v[pl.ds(pos + 3 * PG, 16)] = w1 + 128
                wt_v[2 * ll, pl.ds(lane0, 16)] = 1.0 - ft
                wt_v[2 * ll + 1, pl.ds(lane0, 16)] = ft
                return carry2

            lax.fori_loop(0, L_T_HBM, lvl, 0, unroll=False)
            return carry

        lax.fori_loop(0, PG // 16, grp, 0, unroll=False)

    def reduce_spatial_q(q, vsp_v):
        def grp(g, carry):
            lane0 = g * 16

            def lvl(ll, carry2):
                l = q * LVL_Q + ll
                acc0 = jnp.zeros((16,), jnp.float32)
                acc1 = jnp.zeros((16,), jnp.float32)
                for c in range(8):
                    wv = wsp_v[q * 48 + ll * 8 + c, pl.ds(lane0, 16)]
                    pos = ((ll * 8 + c) * 2) * PG + lane0
                    acc0 = acc0 + vsp_v[pl.ds(pos, 16)] * wv
                    acc1 = acc1 + vsp_v[pl.ds(pos + PG, 16)] * wv
                f_v[2 * l, pl.ds(lane0, 16)] = acc0
                f_v[2 * l + 1, pl.ds(lane0, 16)] = acc1
                return carry2

            lax.fori_loop(0, LVL_Q, lvl, 0, unroll=False)
            return carry

        lax.fori_loop(0, PG // 16, grp, 0, unroll=False)

    def reduce_temporal():
        def grp(g, carry):
            lane0 = g * 16

            def lvl(ll, carry2):
                l = L_T_LUT + ll
                w0 = wt_v[2 * ll, pl.ds(lane0, 16)]
                w1 = wt_v[2 * ll + 1, pl.ds(lane0, 16)]
                pos = (4 * ll) * PG + lane0
                a0 = vt_v[pl.ds(pos, 16)] * w0 + vt_v[pl.ds(pos + 2 * PG, 16)] * w1
                a1 = vt_v[pl.ds(pos + PG, 16)] * w0 + vt_v[pl.ds(pos + 3 * PG, 16)] * w1
                f_v[2 * L_SP + 2 * l, pl.ds(lane0, 16)] = a0
                f_v[2 * L_SP + 2 * l + 1, pl.ds(lane0, 16)] = a1
                return carry2

            lax.fori_loop(0, L_T_HBM, lvl, 0, unroll=False)
            return carry

        lax.fori_loop(0, PG // 16, grp, 0, unroll=False)

    def lut_temporal():
        def grp(g, carry):
            lane0 = g * 16
            t = xyzt_v[3, pl.ds(lane0, 16)]

            def lvl(l, carry2):
                res = res_t_s[l]
                wb = lutw_s[l]
                pt = t * res
                i0 = pt.astype(jnp.int32)
                ft = pt - i0.astype(jnp.float32)
                pp = wb + 2 * i0
                v00 = plsc.load_gather(lut_v, [pp])
                v01 = plsc.load_gather(lut_v, [pp + 1])
                v10 = plsc.load_gather(lut_v, [pp + 2])
                v11 = plsc.load_gather(lut_v, [pp + 3])
                w0 = 1.0 - ft
                f_v[2 * L_SP + 2 * l, pl.ds(lane0, 16)] = v00 * w0 + v10 * ft
                f_v[2 * L_SP + 2 * l + 1, pl.ds(lane0, 16)] = v01 * w0 + v11 * ft
                return carry2

            lax.fori_loop(0, L_T_LUT, lvl, 0, unroll=False)
            return carry

        lax.fori_loop(0, PG // 16, grp, 0, unroll=False)

    def stage_and_head(ci):
        pbase = (wid * chunks_per_tile + ci) * PG
        pltpu.sync_copy(xyzt_hbm.at[:, pl.ds(pbase, PG)], xyzt_v)
        hash_temporal()
        pltpu.async_copy(tt_hbm.at[it_v], vt_v, sem_t)
        hash_spatial_q(0, isp_v0)
        pltpu.async_copy(tsp_hbm.at[isp_v0], vsp_v0, sem0)
        hash_spatial_q(1, isp_v1)
        pltpu.async_copy(tsp_hbm.at[isp_v1], vsp_v1, sem1)

    def wait_sp(isp_v, vsp_v, sem):
        pltpu.make_async_copy(tsp_hbm.at[isp_v], vsp_v, sem).wait()

    stage_and_head(0)

    def chunk(ci, carry):
        pbase = (wid * chunks_per_tile + ci) * PG
        not_last = ci + 1 < chunks_per_tile

        wait_sp(isp_v0, vsp_v0, sem0)
        reduce_spatial_q(0, vsp_v0)
        hash_spatial_q(2, isp_v0)
        pltpu.async_copy(tsp_hbm.at[isp_v0], vsp_v0, sem0)

        wait_sp(isp_v1, vsp_v1, sem1)
        reduce_spatial_q(1, vsp_v1)
        hash_spatial_q(3, isp_v1)
        pltpu.async_copy(tsp_hbm.at[isp_v1], vsp_v1, sem1)

        lut_temporal()
        pltpu.make_async_copy(tt_hbm.at[it_v], vt_v, sem_t).wait()
        reduce_temporal()

        @pl.when(not_last)
        def _():
            pbase2 = pbase + PG
            pltpu.sync_copy(xyzt_hbm.at[:, pl.ds(pbase2, PG)], xyzt_v)
            hash_temporal()
            pltpu.async_copy(tt_hbm.at[it_v], vt_v, sem_t)

        wait_sp(isp_v0, vsp_v0, sem0)
        reduce_spatial_q(2, vsp_v0)

        @pl.when(not_last)
        def _():
            hash_spatial_q(0, isp_v0)
            pltpu.async_copy(tsp_hbm.at[isp_v0], vsp_v0, sem0)

        wait_sp(isp_v1, vsp_v1, sem1)
        reduce_spatial_q(3, vsp_v1)

        @pl.when(not_last)
        def _():
            hash_spatial_q(1, isp_v1)
            pltpu.async_copy(tsp_hbm.at[isp_v1], vsp_v1, sem1)

        pltpu.sync_copy(f_v, feats_hbm.at[:, pl.ds(pbase, PG)])
        return carry

    lax.fori_loop(0, chunks_per_tile, chunk, 0, unroll=False)


def kernel(coords, spatial_table, temporal_table, W1, b1, W2, b2, W3, b3):
    B = coords.shape[0]
    BH = B // NSPLIT
    assert B % (NSPLIT * PG * NW) == 0

    coords_t = coords.T

    tsp = (spatial_table.reshape(L_SP, TBL // 128, 128, FDIM)
           .transpose(0, 1, 3, 2).reshape(L_SP * TBL * FDIM))
    tt = (temporal_table.reshape(L_T, TBL // 128, 128, FDIM)
          .transpose(0, 1, 3, 2).reshape(L_T * TBL * FDIM))

    prep = pl.pallas_call(
        _prep_body,
        grid=(BH // 512,),
        in_specs=[pl.BlockSpec((4, 512), lambda i: (0, i))],
        out_specs=pl.BlockSpec((4, 512), lambda i: (0, i)),
        out_shape=jax.ShapeDtypeStruct((4, BH), jnp.float32),
    )

    mesh = plsc.VectorSubcoreMesh(core_axis_name="c", subcore_axis_name="s")
    enc = pl.kernel(
        _sc_body,
        out_type=jax.ShapeDtypeStruct((D_ENC, BH), jnp.float32),
        mesh=mesh,
        scratch_types=[
            pltpu.VMEM((NSPQ,), jnp.int32),
            pltpu.VMEM((NSPQ,), jnp.int32),
            pltpu.VMEM((NSPQ,), jnp.float32),
            pltpu.VMEM((NSPQ,), jnp.float32),
            pltpu.VMEM((L_SP * 8, PG), jnp.float32),
            pltpu.VMEM((NTH,), jnp.int32),
            pltpu.VMEM((NTH,), jnp.float32),
            pltpu.VMEM((L_T_HBM * 2, PG), jnp.float32),
            pltpu.VMEM((D_ENC, PG), jnp.float32),
            pltpu.VMEM((4, PG), jnp.float32),
            pltpu.VMEM((LUT_WORDS,), jnp.float32),
            pltpu.SMEM((L_SP,), jnp.float32),
            pltpu.SMEM((L_T,), jnp.float32),
            pltpu.SMEM((L_T_LUT,), jnp.int32),
            pltpu.SemaphoreType.DMA,
            pltpu.SemaphoreType.DMA,
            pltpu.SemaphoreType.DMA,
        ],
        compiler_params=pltpu.CompilerParams(needs_layout_passes=False),
    )

    mlp = pl.pallas_call(
        _mlp_body,
        grid=(BH // 512,),
        in_specs=[
            pl.BlockSpec((D_ENC, 512), lambda i: (0, i)),
            pl.BlockSpec((D_ENC, 256), lambda i: (0, 0)),
            pl.BlockSpec((256,), lambda i: (0,)),
            pl.BlockSpec((256, 256), lambda i: (0, 0)),
            pl.BlockSpec((256,), lambda i: (0,)),
            pl.BlockSpec((256, 3), lambda i: (0, 0)),
            pl.BlockSpec((3,), lambda i: (0,)),
        ],
        out_specs=pl.BlockSpec((3, 512), lambda i: (0, i)),
        out_shape=jax.ShapeDtypeStruct((3, BH), jnp.float32),
    )

    outs = []
    for hb in range(NSPLIT):
        xyzt = prep(lax.slice_in_dim(coords_t, hb * BH, (hb + 1) * BH, axis=1))
        feats = enc(xyzt, tsp, tt)
        outs.append(mlp(feats, W1, b1, W2, b2, W3, b3))
    return jnp.concatenate([o.T for o in outs], axis=0)
